# baseline (speedup 1.0000x reference)
import functools

import jax
import jax.numpy as jnp
from jax import lax
from jax.experimental import pallas as pl
from jax.experimental.pallas import tpu as pltpu
from jax.experimental.pallas import tpu_sc as plsc

BATCH = 16384
D = 16
N_ROWS = 1_000_000
NB = BATCH // 128
NW = 32
ROWS_PER_W = NB // NW

_BT = 2048


_TC = 256
_MAIN_GROUPS = 3904
_GPW = _MAIN_GROUPS // NW


def _make_transpose():
    mesh = plsc.VectorSubcoreMesh(core_axis_name="c", subcore_axis_name="s")
    outt = jax.ShapeDtypeStruct((N_ROWS * D // 128, 128), jnp.float32)

    @functools.partial(
        pl.kernel,
        mesh=mesh,
        out_type=[outt, outt],
        compiler_params=pltpu.CompilerParams(needs_layout_passes=False),
        scratch_types=[
            pltpu.VMEM((4, D, _TC), jnp.float32),
            pltpu.VMEM((4, _TC * D // 128, 128), jnp.float32),
            pltpu.SemaphoreType.DMA,
            pltpu.SemaphoreType.DMA,
        ],
    )
    def transpose(ut_t, it_t, tail_u, tail_i, out_u, out_i,
                  ibuf, obuf, sem_in, sem_out):
        wid = lax.axis_index("s") * 2 + lax.axis_index("c")
        iota = lax.iota(jnp.int32, 16)
        n_my = _GPW + jnp.where(wid <= 1, 1, 0).astype(jnp.int32)

        def run_table(src, dst):
            def issue_in(g, b):
                off = pl.multiple_of(g * _TC, 128)
                pltpu.async_copy(src.at[:, pl.ds(off, _TC)],
                                 ibuf.at[b], sem_in)

            def wait_in(b):
                pltpu.make_async_copy(src.at[:, pl.ds(0, _TC)],
                                      ibuf.at[b], sem_in).wait()

            def wait_out(b):
                pltpu.make_async_copy(obuf.at[b],
                                      dst.at[pl.ds(0, _TC * D // 128), :],
                                      sem_out).wait()

            issue_in(wid, 0)
            issue_in(wid + NW, 1)
            issue_in(wid + NW * 2, 2)

            def body(t, carry):
                b = t % 4

                @pl.when(t + 3 < n_my)
                def _():
                    issue_in(wid + NW * (t + 3), (t + 3) % 4)

                wait_in(b)

                @pl.when(t >= 4)
                def _():
                    wait_out(b)

                ib = ibuf.at[b]
                for r0 in range(0, _TC, 16):
                    vals = [
                        plsc.load_gather(
                            ib, [iota, jnp.full((16,), r0 + j, jnp.int32)])
                        for j in range(16)
                    ]
                    for j in range(16):
                        r = r0 + j
                        obuf[b, r // 8, (r % 8) * D:(r % 8 + 1) * D] = vals[j]
                row_off = pl.multiple_of((wid + NW * t) * (_TC * D // 128), 8)
                pltpu.async_copy(obuf.at[b],
                                 dst.at[pl.ds(row_off, _TC * D // 128), :],
                                 sem_out)
                return carry

            lax.fori_loop(0, n_my, body, 0)
            for b in range(4):
                wait_out(b)

        run_table(ut_t, out_u)
        run_table(it_t, out_i)

        @pl.when(wid == 1)
        def _():
            for src, dst in ((tail_u, out_u), (tail_i, out_i)):
                pltpu.sync_copy(src, obuf.at[0].at[pl.ds(0, 8), :])
                pltpu.sync_copy(obuf.at[0].at[pl.ds(0, 8), :],
                                dst.at[pl.ds(N_ROWS * D // 128 - 8, 8), :])

    return transpose


_transpose_tables = _make_transpose()


def _make_gather():
    mesh = plsc.VectorSubcoreMesh(core_axis_name="c", subcore_axis_name="s")
    out3 = jax.ShapeDtypeStruct((NB, 128, D), jnp.float32)

    @functools.partial(
        pl.kernel,
        mesh=mesh,
        out_type=[out3, out3, out3],
        compiler_params=pltpu.CompilerParams(use_tc_tiling_on_sc=False),
        scratch_types=[
            pltpu.VMEM((ROWS_PER_W, 128), jnp.int32),
            pltpu.VMEM((ROWS_PER_W, 128), jnp.int32),
            pltpu.VMEM((ROWS_PER_W, 128), jnp.int32),
            pltpu.VMEM((ROWS_PER_W, 128, D), jnp.float32),
            pltpu.VMEM((ROWS_PER_W, 128, D), jnp.float32),
            pltpu.VMEM((ROWS_PER_W, 128, D), jnp.float32),
            pltpu.SemaphoreType.DMA,
        ],
    )
    def gather(u_idx_hbm, i_idx_hbm, r_idx_hbm, ut_hbm, it_hbm, rt_hbm,
               out_u, out_i, out_r,
               uix, iix, rix, urow, irow, rrow, sem):
        wid = lax.axis_index("s") * 2 + lax.axis_index("c")
        base = wid * ROWS_PER_W
        pltpu.sync_copy(u_idx_hbm.at[pl.ds(base, ROWS_PER_W), :], uix)
        pltpu.sync_copy(i_idx_hbm.at[pl.ds(base, ROWS_PER_W), :], iix)
        pltpu.sync_copy(r_idx_hbm.at[pl.ds(base, ROWS_PER_W), :], rix)
        copies = []
        for c in range(ROWS_PER_W):
            copies.append(pltpu.async_copy(ut_hbm.at[uix.at[c]], urow.at[c], sem))
            copies.append(pltpu.async_copy(it_hbm.at[iix.at[c]], irow.at[c], sem))
            copies.append(pltpu.async_copy(rt_hbm.at[rix.at[c]], rrow.at[c], sem))
        for cp in copies:
            cp.wait()
        pltpu.sync_copy(urow, out_u.at[pl.ds(base, ROWS_PER_W)])
        pltpu.sync_copy(irow, out_i.at[pl.ds(base, ROWS_PER_W)])
        pltpu.sync_copy(rrow, out_r.at[pl.ds(base, ROWS_PER_W)])

    return gather


_gather = _make_gather()

_BM = 2048


def _mlp_body(u_ref, i_ref, r_ref, w1_ref, b1_ref, w2_ref, b2_ref, out_ref):
    w1 = w1_ref[...]
    dn = (((1,), (1,)), ((), ()))
    h = lax.dot_general(u_ref[...], w1[:, 0:16], dn,
                        preferred_element_type=jnp.float32)
    h += lax.dot_general(i_ref[...], w1[:, 16:32], dn,
                         preferred_element_type=jnp.float32)
    h += lax.dot_general(r_ref[...], w1[:, 32:48], dn,
                         preferred_element_type=jnp.float32)
    h += b1_ref[...]
    h = jnp.maximum(h, 0.0)
    p = jnp.sum(h * w2_ref[...], axis=1, keepdims=True)
    p += b2_ref[0, 0]
    out_ref[...] = 10.0 / (1.0 + jnp.exp(-p))


@jax.jit
def _mlp(emb_u, emb_i, emb_r, W1, b1, W2, b2):
    grid = (BATCH // _BM,)
    return pl.pallas_call(
        _mlp_body,
        grid=grid,
        in_specs=[
            pl.BlockSpec((_BM, D), lambda i: (i, 0)),
            pl.BlockSpec((_BM, D), lambda i: (i, 0)),
            pl.BlockSpec((_BM, D), lambda i: (i, 0)),
            pl.BlockSpec((128, 48), lambda i: (0, 0)),
            pl.BlockSpec((1, 128), lambda i: (0, 0)),
            pl.BlockSpec((1, 128), lambda i: (0, 0)),
            pl.BlockSpec((1, 1), lambda i: (0, 0)),
        ],
        out_specs=pl.BlockSpec((_BM, 1), lambda i: (i, 0)),
        out_shape=jax.ShapeDtypeStruct((BATCH, 1), jnp.float32),
    )(emb_u, emb_i, emb_r, W1, b1, W2, b2)


def kernel(X, user_table, item_table, rating_table, W1, b1, W2, b2):
    Xi = X.astype(jnp.int32)
    u_idx = Xi[:, 0].reshape(NB, 128)
    i_idx = Xi[:, 1].reshape(NB, 128)
    r_idx = Xi[:, 2].reshape(NB, 128)
    ut_d, it_d = _transpose_tables(user_table.T, item_table.T,
                                   user_table[N_ROWS - 64:, :].reshape(8, 128),
                                   item_table[N_ROWS - 64:, :].reshape(8, 128))
    ut_rm = ut_d.reshape(N_ROWS, D)
    it_rm = it_d.reshape(N_ROWS, D)
    eu, ei, er = _gather(u_idx, i_idx, r_idx, ut_rm, it_rm, rating_table)
    emb_u = eu.reshape(BATCH, D)
    emb_i = ei.reshape(BATCH, D)
    emb_r = er.reshape(BATCH, D)
    return _mlp(emb_u, emb_i, emb_r, W1,
                b1.reshape(1, 128), W2, b2.reshape(1, 1))


# speedup vs baseline: 1.0418x; 1.0418x over previous
import functools

import jax
import jax.numpy as jnp
from jax import lax
from jax.experimental import pallas as pl
from jax.experimental.pallas import tpu as pltpu
from jax.experimental.pallas import tpu_sc as plsc

BATCH = 16384
D = 16
N_ROWS = 1_000_000
NB = BATCH // 128
NW = 32
ROWS_PER_W = NB // NW

_BT = 2048


_TC = 128
_MAIN_GROUPS = 7808
_GPW = _MAIN_GROUPS // NW


def _make_transpose():
    mesh = plsc.VectorSubcoreMesh(core_axis_name="c", subcore_axis_name="s")
    outt = jax.ShapeDtypeStruct((N_ROWS * D // 128, 128), jnp.float32)

    @functools.partial(
        pl.kernel,
        mesh=mesh,
        out_type=[outt, outt],
        compiler_params=pltpu.CompilerParams(needs_layout_passes=False),
        scratch_types=[
            pltpu.VMEM((4, D, _TC), jnp.float32),
            pltpu.VMEM((4, _TC * D // 128, 128), jnp.float32),
            pltpu.SemaphoreType.DMA,
            pltpu.SemaphoreType.DMA,
        ],
    )
    def transpose(ut_t, it_t, tail_u, tail_i, out_u, out_i,
                  ibuf, obuf, sem_in, sem_out):
        wid = lax.axis_index("s") * 2 + lax.axis_index("c")
        iota = lax.iota(jnp.int32, 16)
        n_my = _GPW + jnp.where(wid <= 3, 1, 0).astype(jnp.int32)

        def run_table(src, dst):
            def issue_in(g, b):
                off = pl.multiple_of(g * _TC, 128)
                pltpu.async_copy(src.at[:, pl.ds(off, _TC)],
                                 ibuf.at[b], sem_in)

            def wait_in(b):
                pltpu.make_async_copy(src.at[:, pl.ds(0, _TC)],
                                      ibuf.at[b], sem_in).wait()

            def wait_out(b):
                pltpu.make_async_copy(obuf.at[b],
                                      dst.at[pl.ds(0, _TC * D // 128), :],
                                      sem_out).wait()

            issue_in(wid, 0)
            issue_in(wid + NW, 1)
            issue_in(wid + NW * 2, 2)

            def body(t, carry):
                b = t % 4

                @pl.when(t + 3 < n_my)
                def _():
                    issue_in(wid + NW * (t + 3), (t + 3) % 4)

                wait_in(b)

                @pl.when(t >= 4)
                def _():
                    wait_out(b)

                ib = ibuf.at[b]
                for r0 in range(0, _TC, 16):
                    vals = [
                        plsc.load_gather(
                            ib, [iota, jnp.full((16,), r0 + j, jnp.int32)])
                        for j in range(16)
                    ]
                    for j in range(16):
                        r = r0 + j
                        obuf[b, r // 8, (r % 8) * D:(r % 8 + 1) * D] = vals[j]
                row_off = pl.multiple_of((wid + NW * t) * (_TC * D // 128), 8)
                pltpu.async_copy(obuf.at[b],
                                 dst.at[pl.ds(row_off, _TC * D // 128), :],
                                 sem_out)
                return carry

            lax.fori_loop(0, n_my, body, 0)
            for b in range(4):
                wait_out(b)

        run_table(ut_t, out_u)
        run_table(it_t, out_i)

        @pl.when(wid == 1)
        def _():
            for src, dst in ((tail_u, out_u), (tail_i, out_i)):
                pltpu.sync_copy(src, obuf.at[0].at[pl.ds(0, 8), :])
                pltpu.sync_copy(obuf.at[0].at[pl.ds(0, 8), :],
                                dst.at[pl.ds(N_ROWS * D // 128 - 8, 8), :])

    return transpose


_transpose_tables = _make_transpose()


def _make_gather():
    mesh = plsc.VectorSubcoreMesh(core_axis_name="c", subcore_axis_name="s")
    out3 = jax.ShapeDtypeStruct((NB, 128, D), jnp.float32)

    @functools.partial(
        pl.kernel,
        mesh=mesh,
        out_type=[out3, out3, out3],
        compiler_params=pltpu.CompilerParams(use_tc_tiling_on_sc=False),
        scratch_types=[
            pltpu.VMEM((ROWS_PER_W, 128), jnp.int32),
            pltpu.VMEM((ROWS_PER_W, 128), jnp.int32),
            pltpu.VMEM((ROWS_PER_W, 128), jnp.int32),
            pltpu.VMEM((ROWS_PER_W, 128, D), jnp.float32),
            pltpu.VMEM((ROWS_PER_W, 128, D), jnp.float32),
            pltpu.VMEM((ROWS_PER_W, 128, D), jnp.float32),
            pltpu.SemaphoreType.DMA,
        ],
    )
    def gather(u_idx_hbm, i_idx_hbm, r_idx_hbm, ut_hbm, it_hbm, rt_hbm,
               out_u, out_i, out_r,
               uix, iix, rix, urow, irow, rrow, sem):
        wid = lax.axis_index("s") * 2 + lax.axis_index("c")
        base = wid * ROWS_PER_W
        pltpu.sync_copy(u_idx_hbm.at[pl.ds(base, ROWS_PER_W), :], uix)
        pltpu.sync_copy(i_idx_hbm.at[pl.ds(base, ROWS_PER_W), :], iix)
        pltpu.sync_copy(r_idx_hbm.at[pl.ds(base, ROWS_PER_W), :], rix)
        copies = []
        for c in range(ROWS_PER_W):
            copies.append(pltpu.async_copy(ut_hbm.at[uix.at[c]], urow.at[c], sem))
            copies.append(pltpu.async_copy(it_hbm.at[iix.at[c]], irow.at[c], sem))
            copies.append(pltpu.async_copy(rt_hbm.at[rix.at[c]], rrow.at[c], sem))
        for cp in copies:
            cp.wait()
        pltpu.sync_copy(urow, out_u.at[pl.ds(base, ROWS_PER_W)])
        pltpu.sync_copy(irow, out_i.at[pl.ds(base, ROWS_PER_W)])
        pltpu.sync_copy(rrow, out_r.at[pl.ds(base, ROWS_PER_W)])

    return gather


_gather = _make_gather()

_BM = 2048


def _mlp_body(u_ref, i_ref, r_ref, w1_ref, b1_ref, w2_ref, b2_ref, out_ref):
    w1 = w1_ref[...]
    dn = (((1,), (1,)), ((), ()))
    h = lax.dot_general(u_ref[...], w1[:, 0:16], dn,
                        preferred_element_type=jnp.float32)
    h += lax.dot_general(i_ref[...], w1[:, 16:32], dn,
                         preferred_element_type=jnp.float32)
    h += lax.dot_general(r_ref[...], w1[:, 32:48], dn,
                         preferred_element_type=jnp.float32)
    h += b1_ref[...]
    h = jnp.maximum(h, 0.0)
    p = jnp.sum(h * w2_ref[...], axis=1, keepdims=True)
    p += b2_ref[0, 0]
    out_ref[...] = 10.0 / (1.0 + jnp.exp(-p))


@jax.jit
def _mlp(emb_u, emb_i, emb_r, W1, b1, W2, b2):
    grid = (BATCH // _BM,)
    return pl.pallas_call(
        _mlp_body,
        grid=grid,
        in_specs=[
            pl.BlockSpec((_BM, D), lambda i: (i, 0)),
            pl.BlockSpec((_BM, D), lambda i: (i, 0)),
            pl.BlockSpec((_BM, D), lambda i: (i, 0)),
            pl.BlockSpec((128, 48), lambda i: (0, 0)),
            pl.BlockSpec((1, 128), lambda i: (0, 0)),
            pl.BlockSpec((1, 128), lambda i: (0, 0)),
            pl.BlockSpec((1, 1), lambda i: (0, 0)),
        ],
        out_specs=pl.BlockSpec((_BM, 1), lambda i: (i, 0)),
        out_shape=jax.ShapeDtypeStruct((BATCH, 1), jnp.float32),
    )(emb_u, emb_i, emb_r, W1, b1, W2, b2)


def kernel(X, user_table, item_table, rating_table, W1, b1, W2, b2):
    Xi = X.astype(jnp.int32)
    u_idx = Xi[:, 0].reshape(NB, 128)
    i_idx = Xi[:, 1].reshape(NB, 128)
    r_idx = Xi[:, 2].reshape(NB, 128)
    ut_d, it_d = _transpose_tables(user_table.T, item_table.T,
                                   user_table[N_ROWS - 64:, :].reshape(8, 128),
                                   item_table[N_ROWS - 64:, :].reshape(8, 128))
    ut_rm = ut_d.reshape(N_ROWS, D)
    it_rm = it_d.reshape(N_ROWS, D)
    eu, ei, er = _gather(u_idx, i_idx, r_idx, ut_rm, it_rm, rating_table)
    emb_u = eu.reshape(BATCH, D)
    emb_i = ei.reshape(BATCH, D)
    emb_r = er.reshape(BATCH, D)
    return _mlp(emb_u, emb_i, emb_r, W1,
                b1.reshape(1, 128), W2, b2.reshape(1, 1))
